# initial kernel scaffold (unmeasured)
import functools

import jax
import jax.numpy as jnp
from jax import lax
from jax.experimental import pallas as pl
from jax.experimental.pallas import tpu as pltpu

N_DEV = 32
GELU_C = 0.7978845608028654


def _gelu(y):
    return 0.5 * y * (1.0 + jnp.tanh(GELU_C * (y + 0.044715 * y * y * y)))


def kernel(x, w_mat):
    m, k_per = x.shape
    k_per2, n = w_mat.shape
    assert k_per == k_per2
    m_chunk = m // N_DEV

    def body(x_ref, w_ref, out_ref, recv_buf, send_buf, send_sem, recv_sems,
             credit_sems):
        my = lax.axis_index("i")
        left = lax.rem(my - 1 + N_DEV, N_DEV)
        right = lax.rem(my + 1, N_DEV)

        def partial_chunk(c):
            xc = x_ref[pl.ds(c * m_chunk, m_chunk), :]
            return jnp.dot(xc, w_ref[:, :], preferred_element_type=jnp.float32)

        c0 = lax.rem(my - 1 + N_DEV, N_DEV)
        send_buf[:, :] = partial_chunk(c0)

        for s in range(N_DEV - 1):
            slot = s % 2
            if s >= 2:
                pl.semaphore_wait(credit_sems.at[slot], 1)
            rdma = pltpu.make_async_remote_copy(
                src_ref=send_buf,
                dst_ref=recv_buf.at[slot],
                send_sem=send_sem,
                recv_sem=recv_sems.at[slot],
                device_id=(right,),
                device_id_type=pl.DeviceIdType.MESH,
            )
            rdma.start()
            rdma.wait()

            c = lax.rem(my - 2 - s + 2 * N_DEV, N_DEV)
            val = recv_buf[slot, :, :] + partial_chunk(c)
            if s < N_DEV - 2:
                send_buf[:, :] = val
            else:
                out_ref[:, :] = _gelu(val)
            if s <= N_DEV - 3:
                pl.semaphore_signal(
                    credit_sems.at[slot], inc=1,
                    device_id=(left,),
                    device_id_type=pl.DeviceIdType.MESH,
                )

    out_shape = jax.ShapeDtypeStruct((m_chunk, n), jnp.float32)
    return pl.pallas_call(
        body,
        out_shape=out_shape,
        in_specs=[
            pl.BlockSpec(memory_space=pltpu.VMEM),
            pl.BlockSpec(memory_space=pltpu.VMEM),
        ],
        out_specs=pl.BlockSpec(memory_space=pltpu.VMEM),
        scratch_shapes=[
            pltpu.VMEM((2, m_chunk, n), jnp.float32),
            pltpu.VMEM((m_chunk, n), jnp.float32),
            pltpu.SemaphoreType.DMA,
            pltpu.SemaphoreType.DMA((2,)),
            pltpu.SemaphoreType.REGULAR((2,)),
        ],
        compiler_params=pltpu.CompilerParams(collective_id=0),
    )(x, w_mat)


# baseline (device time: 243915 ns/iter reference)
import functools

import jax
import jax.numpy as jnp
from jax import lax
from jax.experimental import pallas as pl
from jax.experimental.pallas import tpu as pltpu

N_DEV = 32
GELU_C = 0.7978845608028654


def _gelu(y):
    return 0.5 * y * (1.0 + jnp.tanh(GELU_C * (y + 0.044715 * y * y * y)))


def kernel(x, w_mat):
    m, k_per = x.shape
    k_per2, n = w_mat.shape
    assert k_per == k_per2
    m_chunk = m // N_DEV

    def body(x_ref, w_ref, out_ref, recv_buf, send_buf, send_sem, recv_sems,
             credit_sems):
        my = lax.axis_index("i")
        left = lax.rem(my - 1 + N_DEV, N_DEV)
        right = lax.rem(my + 1, N_DEV)

        barrier_sem = pltpu.get_barrier_semaphore()
        for nbr in (left, right):
            pl.semaphore_signal(
                barrier_sem, inc=1,
                device_id=(nbr,), device_id_type=pl.DeviceIdType.MESH,
            )
        pl.semaphore_wait(barrier_sem, 2)

        def partial_chunk(c):
            xc = x_ref[pl.ds(c * m_chunk, m_chunk), :]
            return jnp.dot(xc, w_ref[:, :], preferred_element_type=jnp.float32)

        c0 = lax.rem(my - 1 + N_DEV, N_DEV)
        send_buf[:, :] = partial_chunk(c0)

        for s in range(N_DEV - 1):
            slot = s % 2
            if s >= 2:
                pl.semaphore_wait(credit_sems.at[slot], 1)
            rdma = pltpu.make_async_remote_copy(
                src_ref=send_buf,
                dst_ref=recv_buf.at[slot],
                send_sem=send_sem,
                recv_sem=recv_sems.at[slot],
                device_id=(right,),
                device_id_type=pl.DeviceIdType.MESH,
            )
            rdma.start()
            rdma.wait()

            c = lax.rem(my - 2 - s + 2 * N_DEV, N_DEV)
            val = recv_buf[slot, :, :] + partial_chunk(c)
            if s < N_DEV - 2:
                send_buf[:, :] = val
            else:
                out_ref[:, :] = _gelu(val)
            if s <= N_DEV - 4:
                pl.semaphore_signal(
                    credit_sems.at[slot], inc=1,
                    device_id=(left,),
                    device_id_type=pl.DeviceIdType.MESH,
                )

    out_shape = jax.ShapeDtypeStruct((m_chunk, n), jnp.float32)
    return pl.pallas_call(
        body,
        out_shape=out_shape,
        in_specs=[
            pl.BlockSpec(memory_space=pltpu.VMEM),
            pl.BlockSpec(memory_space=pltpu.VMEM),
        ],
        out_specs=pl.BlockSpec(memory_space=pltpu.VMEM),
        scratch_shapes=[
            pltpu.VMEM((2, m_chunk, n), jnp.float32),
            pltpu.VMEM((m_chunk, n), jnp.float32),
            pltpu.SemaphoreType.DMA,
            pltpu.SemaphoreType.DMA((2,)),
            pltpu.SemaphoreType.REGULAR((2,)),
        ],
        compiler_params=pltpu.CompilerParams(collective_id=0),
    )(x, w_mat)


# device time: 163737 ns/iter; 1.4897x vs baseline; 1.4897x over previous
import functools

import jax
import jax.numpy as jnp
from jax import lax
from jax.experimental import pallas as pl
from jax.experimental.pallas import tpu as pltpu

N_DEV = 32
GELU_C = 0.7978845608028654


def _gelu(y):
    return 0.5 * y * (1.0 + jnp.tanh(GELU_C * (y + 0.044715 * y * y * y)))


def kernel(x, w_mat):
    m, k_per = x.shape
    k_per2, n = w_mat.shape
    assert k_per == k_per2
    m_chunk = m // N_DEV

    def body(x_ref, w_ref, out_ref, recv_buf, send_buf, send_sem, recv_sems,
             credit_sems):
        my = lax.axis_index("i")
        left = lax.rem(my - 1 + N_DEV, N_DEV)
        right = lax.rem(my + 1, N_DEV)

        barrier_sem = pltpu.get_barrier_semaphore()
        for nbr in (left, right):
            pl.semaphore_signal(
                barrier_sem, inc=1,
                device_id=(nbr,), device_id_type=pl.DeviceIdType.MESH,
            )
        pl.semaphore_wait(barrier_sem, 2)

        def partial_chunk(c):
            xc = x_ref[pl.ds(c * m_chunk, m_chunk), :]
            return jnp.dot(xc, w_ref[:, :], preferred_element_type=jnp.float32)

        c0 = lax.rem(my - 1 + N_DEV, N_DEV)
        send_buf[:, :] = partial_chunk(c0).astype(jnp.bfloat16)

        for s in range(N_DEV - 1):
            slot = s % 2
            if s >= 2:
                pl.semaphore_wait(credit_sems.at[slot], 1)
            rdma = pltpu.make_async_remote_copy(
                src_ref=send_buf,
                dst_ref=recv_buf.at[slot],
                send_sem=send_sem,
                recv_sem=recv_sems.at[slot],
                device_id=(right,),
                device_id_type=pl.DeviceIdType.MESH,
            )
            rdma.start()

            c = lax.rem(my - 2 - s + 2 * N_DEV, N_DEV)
            p = partial_chunk(c)
            rdma.wait()

            val = recv_buf[slot, :, :].astype(jnp.float32) + p
            if s < N_DEV - 2:
                send_buf[:, :] = val.astype(jnp.bfloat16)
            else:
                out_ref[:, :] = _gelu(val)
            if s <= N_DEV - 4:
                pl.semaphore_signal(
                    credit_sems.at[slot], inc=1,
                    device_id=(left,),
                    device_id_type=pl.DeviceIdType.MESH,
                )

    out_shape = jax.ShapeDtypeStruct((m_chunk, n), jnp.float32)
    return pl.pallas_call(
        body,
        out_shape=out_shape,
        in_specs=[
            pl.BlockSpec(memory_space=pltpu.VMEM),
            pl.BlockSpec(memory_space=pltpu.VMEM),
        ],
        out_specs=pl.BlockSpec(memory_space=pltpu.VMEM),
        scratch_shapes=[
            pltpu.VMEM((2, m_chunk, n), jnp.bfloat16),
            pltpu.VMEM((m_chunk, n), jnp.bfloat16),
            pltpu.SemaphoreType.DMA,
            pltpu.SemaphoreType.DMA((2,)),
            pltpu.SemaphoreType.REGULAR((2,)),
        ],
        compiler_params=pltpu.CompilerParams(collective_id=0),
    )(x, w_mat)


# device time: 121304 ns/iter; 2.0108x vs baseline; 1.3498x over previous
import jax
import jax.numpy as jnp
from jax import lax
from jax.experimental import pallas as pl
from jax.experimental.pallas import tpu as pltpu

N_DEV = 32
N_STEP_R = 16
N_STEP_L = 15
GELU_C = 0.7978845608028654


def _gelu(y):
    return 0.5 * y * (1.0 + jnp.tanh(GELU_C * (y + 0.044715 * y * y * y)))


def kernel(x, w_mat):
    m, k_per = x.shape
    k_per2, n = w_mat.shape
    assert k_per == k_per2
    m_chunk = m // N_DEV

    def body(x_ref, w_ref, out_ref,
             recv_r, send_r, recv_l, send_l,
             send_sem_r, recv_sems_r, send_sem_l, recv_sems_l,
             credit_r, credit_l):
        my = lax.axis_index("i")
        left = lax.rem(my - 1 + N_DEV, N_DEV)
        right = lax.rem(my + 1, N_DEV)

        barrier_sem = pltpu.get_barrier_semaphore()
        for nbr in (left, right):
            pl.semaphore_signal(
                barrier_sem, inc=1,
                device_id=(nbr,), device_id_type=pl.DeviceIdType.MESH,
            )
        pl.semaphore_wait(barrier_sem, 2)

        def partial_chunk(c):
            xc = x_ref[pl.ds(c * m_chunk, m_chunk), :]
            return jnp.dot(xc, w_ref[:, :], preferred_element_type=jnp.float32)

        def chunk_id(off):
            return lax.rem(my + off + 2 * N_DEV, N_DEV)

        send_r[:, :] = partial_chunk(chunk_id(-N_STEP_R)).astype(jnp.bfloat16)
        send_l[:, :] = partial_chunk(chunk_id(-N_STEP_L)).astype(jnp.bfloat16)

        final_l = None
        for s in range(N_STEP_R):
            slot = s % 2
            l_active = s < N_STEP_L

            if s >= 2:
                pl.semaphore_wait(credit_r.at[slot], 1)
                if l_active:
                    pl.semaphore_wait(credit_l.at[slot], 1)

            rdma_r = pltpu.make_async_remote_copy(
                src_ref=send_r,
                dst_ref=recv_r.at[slot],
                send_sem=send_sem_r,
                recv_sem=recv_sems_r.at[slot],
                device_id=(right,),
                device_id_type=pl.DeviceIdType.MESH,
            )
            rdma_r.start()
            if l_active:
                rdma_l = pltpu.make_async_remote_copy(
                    src_ref=send_l,
                    dst_ref=recv_l.at[slot],
                    send_sem=send_sem_l,
                    recv_sem=recv_sems_l.at[slot],
                    device_id=(left,),
                    device_id_type=pl.DeviceIdType.MESH,
                )
                rdma_l.start()

            p_r = partial_chunk(chunk_id(-17 - s if s < N_STEP_R - 1 else 0))
            if s < N_STEP_L - 1:
                p_l = partial_chunk(chunk_id(-14 + s))

            rdma_r.wait()
            val_r = recv_r[slot, :, :].astype(jnp.float32) + p_r
            if s < N_STEP_R - 1:
                send_r[:, :] = val_r.astype(jnp.bfloat16)
            if s <= N_STEP_R - 3:
                pl.semaphore_signal(
                    credit_r.at[slot], inc=1,
                    device_id=(left,), device_id_type=pl.DeviceIdType.MESH,
                )

            if l_active:
                rdma_l.wait()
                if s < N_STEP_L - 1:
                    val_l = recv_l[slot, :, :].astype(jnp.float32) + p_l
                    send_l[:, :] = val_l.astype(jnp.bfloat16)
                else:
                    final_l = recv_l[slot, :, :].astype(jnp.float32)
                if s <= N_STEP_L - 3:
                    pl.semaphore_signal(
                        credit_l.at[slot], inc=1,
                        device_id=(right,), device_id_type=pl.DeviceIdType.MESH,
                    )

        out_ref[:, :] = _gelu(val_r + final_l)

    out_shape = jax.ShapeDtypeStruct((m_chunk, n), jnp.float32)
    return pl.pallas_call(
        body,
        out_shape=out_shape,
        in_specs=[
            pl.BlockSpec(memory_space=pltpu.VMEM),
            pl.BlockSpec(memory_space=pltpu.VMEM),
        ],
        out_specs=pl.BlockSpec(memory_space=pltpu.VMEM),
        scratch_shapes=[
            pltpu.VMEM((2, m_chunk, n), jnp.bfloat16),
            pltpu.VMEM((m_chunk, n), jnp.bfloat16),
            pltpu.VMEM((2, m_chunk, n), jnp.bfloat16),
            pltpu.VMEM((m_chunk, n), jnp.bfloat16),
            pltpu.SemaphoreType.DMA,
            pltpu.SemaphoreType.DMA((2,)),
            pltpu.SemaphoreType.DMA,
            pltpu.SemaphoreType.DMA((2,)),
            pltpu.SemaphoreType.REGULAR((2,)),
            pltpu.SemaphoreType.REGULAR((2,)),
        ],
        compiler_params=pltpu.CompilerParams(collective_id=0),
    )(x, w_mat)


# device time: 97231 ns/iter; 2.5086x vs baseline; 1.2476x over previous
import jax
import jax.numpy as jnp
from jax import lax
from jax.experimental import pallas as pl
from jax.experimental.pallas import tpu as pltpu

N_DEV = 32
N_STEP_R = 16
N_STEP_L = 15
T = 4
GELU_C = 0.7978845608028654


def _gelu(y):
    return 0.5 * y * (1.0 + jnp.tanh(GELU_C * (y + 0.044715 * y * y * y)))


def kernel(x, w_mat):
    m, k_per = x.shape
    k_per2, n = w_mat.shape
    assert k_per == k_per2
    m_chunk = m // N_DEV
    tw = n // T

    def body(x_ref, w_ref, out_ref,
             recv_r, send_r, recv_l, send_l,
             send_sems_r, recv_sems_r, send_sems_l, recv_sems_l,
             credit_r, credit_l):
        my = lax.axis_index("i")
        left = lax.rem(my - 1 + N_DEV, N_DEV)
        right = lax.rem(my + 1, N_DEV)

        barrier_sem = pltpu.get_barrier_semaphore()
        for nbr in (left, right):
            pl.semaphore_signal(
                barrier_sem, inc=1,
                device_id=(nbr,), device_id_type=pl.DeviceIdType.MESH,
            )
        pl.semaphore_wait(barrier_sem, 2)

        def partial_chunk(c):
            xc = x_ref[pl.ds(c * m_chunk, m_chunk), :]
            return jnp.dot(xc, w_ref[:, :], preferred_element_type=jnp.float32)

        def chunk_id(off):
            return lax.rem(my + off + 2 * N_DEV, N_DEV)

        def make_rdma(flow, slot, t):
            src, dst = (send_r, recv_r) if flow == "r" else (send_l, recv_l)
            ssem, rsem = (
                (send_sems_r, recv_sems_r) if flow == "r"
                else (send_sems_l, recv_sems_l)
            )
            tgt = right if flow == "r" else left
            return pltpu.make_async_remote_copy(
                src_ref=src.at[t],
                dst_ref=dst.at[slot, t],
                send_sem=ssem.at[t],
                recv_sem=rsem.at[slot, t],
                device_id=(tgt,),
                device_id_type=pl.DeviceIdType.MESH,
            )

        p0_r = partial_chunk(chunk_id(-N_STEP_R))
        p0_l = partial_chunk(chunk_id(-N_STEP_L))
        cur_r = [None] * T
        cur_l = [None] * T
        for t in range(T):
            send_r[t, :, :] = p0_r[:, t * tw:(t + 1) * tw].astype(jnp.bfloat16)
            send_l[t, :, :] = p0_l[:, t * tw:(t + 1) * tw].astype(jnp.bfloat16)
            cur_r[t] = make_rdma("r", 0, t)
            cur_r[t].start()
            cur_l[t] = make_rdma("l", 0, t)
            cur_l[t].start()

        final_l = [None] * T
        out_r = [None] * T

        for s in range(N_STEP_R):
            slot = s % 2
            nslot = (s + 1) % 2
            l_active = s < N_STEP_L
            p_r = partial_chunk(chunk_id(-17 - s))
            if l_active and s < N_STEP_L - 1:
                p_l = partial_chunk(chunk_id(-14 + s))

            for t in range(T):
                cs = slice(t * tw, (t + 1) * tw)
                cur_r[t].wait_recv()
                val_r = recv_r[slot, t, :, :].astype(jnp.float32) + p_r[:, cs]
                if s < N_STEP_R - 1:
                    cur_r[t].wait_send()
                    send_r[t, :, :] = val_r.astype(jnp.bfloat16)
                    if s + 1 >= 2:
                        pl.semaphore_wait(credit_r.at[nslot, t], 1)
                    cur_r[t] = make_rdma("r", nslot, t)
                    cur_r[t].start()
                else:
                    out_r[t] = val_r
                if s <= N_STEP_R - 3:
                    pl.semaphore_signal(
                        credit_r.at[slot, t], inc=1,
                        device_id=(left,), device_id_type=pl.DeviceIdType.MESH,
                    )
                if l_active:
                    cur_l[t].wait_recv()
                    if s < N_STEP_L - 1:
                        val_l = (recv_l[slot, t, :, :].astype(jnp.float32)
                                 + p_l[:, cs])
                        cur_l[t].wait_send()
                        send_l[t, :, :] = val_l.astype(jnp.bfloat16)
                        if s + 1 >= 2:
                            pl.semaphore_wait(credit_l.at[nslot, t], 1)
                        cur_l[t] = make_rdma("l", nslot, t)
                        cur_l[t].start()
                    else:
                        final_l[t] = recv_l[slot, t, :, :].astype(jnp.float32)
                    if s <= N_STEP_L - 3:
                        pl.semaphore_signal(
                            credit_l.at[slot, t], inc=1,
                            device_id=(right,),
                            device_id_type=pl.DeviceIdType.MESH,
                        )

        for t in range(T):
            cur_r[t].wait_send()
            cur_l[t].wait_send()
            out_ref[:, t * tw:(t + 1) * tw] = _gelu(out_r[t] + final_l[t])

    out_shape = jax.ShapeDtypeStruct((m_chunk, n), jnp.float32)
    return pl.pallas_call(
        body,
        out_shape=out_shape,
        in_specs=[
            pl.BlockSpec(memory_space=pltpu.VMEM),
            pl.BlockSpec(memory_space=pltpu.VMEM),
        ],
        out_specs=pl.BlockSpec(memory_space=pltpu.VMEM),
        scratch_shapes=[
            pltpu.VMEM((2, T, m_chunk, tw), jnp.bfloat16),
            pltpu.VMEM((T, m_chunk, tw), jnp.bfloat16),
            pltpu.VMEM((2, T, m_chunk, tw), jnp.bfloat16),
            pltpu.VMEM((T, m_chunk, tw), jnp.bfloat16),
            pltpu.SemaphoreType.DMA((T,)),
            pltpu.SemaphoreType.DMA((2, T)),
            pltpu.SemaphoreType.DMA((T,)),
            pltpu.SemaphoreType.DMA((2, T)),
            pltpu.SemaphoreType.REGULAR((2, T)),
            pltpu.SemaphoreType.REGULAR((2, T)),
        ],
        compiler_params=pltpu.CompilerParams(collective_id=0),
    )(x, w_mat)


# device time: 53584 ns/iter; 4.5520x vs baseline; 1.8146x over previous
import jax
import jax.numpy as jnp
from jax import lax
from jax.experimental import pallas as pl
from jax.experimental.pallas import tpu as pltpu

N_DEV = 32
N_STEP_R = 16
N_STEP_L = 15
T = 4
GELU_C = 0.7978845608028654


def _gelu(y):
    return 0.5 * y * (1.0 + jnp.tanh(GELU_C * (y + 0.044715 * y * y * y)))


def kernel(x, w_mat):
    m, k_per = x.shape
    k_per2, n = w_mat.shape
    assert k_per == k_per2
    m_chunk = m // N_DEV
    tw = n // T

    def body(x_ref, w_ref, out_ref,
             recv_r, send_r, recv_l, send_l,
             send_sems_r, recv_sems_r, send_sems_l, recv_sems_l,
             credit_r, credit_l):
        my = lax.axis_index("i")

        def pos_of(idx):
            z = lax.div(idx, 8)
            k = lax.rem(idx, 8)
            y = lax.div(k, 2)
            x = lax.rem(lax.div(k + 1, 2), 2)
            p = z * 4 + jnp.where(lax.rem(z, 2) == 0, y, 3 - y)
            return jnp.where(x == 0, p, 31 - p)

        def log_of(pos):
            in_x0 = pos < 16
            p = jnp.where(in_x0, pos, 31 - pos)
            z = lax.div(p, 4)
            r = lax.rem(p, 4)
            y = jnp.where(lax.rem(z, 2) == 0, r, 3 - r)
            xv = jnp.where(in_x0, 0, 1)
            k = 2 * y + jnp.where(lax.rem(y, 2) == 0, xv, 1 - xv)
            return z * 8 + k

        pos = pos_of(my)
        left = log_of(lax.rem(pos - 1 + N_DEV, N_DEV))
        right = log_of(lax.rem(pos + 1, N_DEV))

        barrier_sem = pltpu.get_barrier_semaphore()
        for nbr in (left, right):
            pl.semaphore_signal(
                barrier_sem, inc=1,
                device_id=(nbr,), device_id_type=pl.DeviceIdType.MESH,
            )
        pl.semaphore_wait(barrier_sem, 2)

        def partial_chunk(c):
            xc = x_ref[pl.ds(c * m_chunk, m_chunk), :]
            return jnp.dot(xc, w_ref[:, :], preferred_element_type=jnp.float32)

        def chunk_id(off):
            return log_of(lax.rem(pos + off + 2 * N_DEV, N_DEV))

        def make_rdma(flow, slot, t):
            src, dst = (send_r, recv_r) if flow == "r" else (send_l, recv_l)
            ssem, rsem = (
                (send_sems_r, recv_sems_r) if flow == "r"
                else (send_sems_l, recv_sems_l)
            )
            tgt = right if flow == "r" else left
            return pltpu.make_async_remote_copy(
                src_ref=src.at[t],
                dst_ref=dst.at[slot, t],
                send_sem=ssem.at[t],
                recv_sem=rsem.at[slot, t],
                device_id=(tgt,),
                device_id_type=pl.DeviceIdType.MESH,
            )

        p0_r = partial_chunk(chunk_id(-N_STEP_R))
        p0_l = partial_chunk(chunk_id(-N_STEP_L))
        cur_r = [None] * T
        cur_l = [None] * T
        for t in range(T):
            send_r[t, :, :] = p0_r[:, t * tw:(t + 1) * tw].astype(jnp.bfloat16)
            send_l[t, :, :] = p0_l[:, t * tw:(t + 1) * tw].astype(jnp.bfloat16)
            cur_r[t] = make_rdma("r", 0, t)
            cur_r[t].start()
            cur_l[t] = make_rdma("l", 0, t)
            cur_l[t].start()

        final_l = [None] * T
        out_r = [None] * T

        for s in range(N_STEP_R):
            slot = s % 2
            nslot = (s + 1) % 2
            l_active = s < N_STEP_L
            p_r = partial_chunk(chunk_id(-17 - s))
            if l_active and s < N_STEP_L - 1:
                p_l = partial_chunk(chunk_id(-14 + s))

            for t in range(T):
                cs = slice(t * tw, (t + 1) * tw)
                cur_r[t].wait_recv()
                val_r = recv_r[slot, t, :, :].astype(jnp.float32) + p_r[:, cs]
                if s < N_STEP_R - 1:
                    cur_r[t].wait_send()
                    send_r[t, :, :] = val_r.astype(jnp.bfloat16)
                    if s + 1 >= 2:
                        pl.semaphore_wait(credit_r.at[nslot, t], 1)
                    cur_r[t] = make_rdma("r", nslot, t)
                    cur_r[t].start()
                else:
                    out_r[t] = val_r
                if s <= N_STEP_R - 3:
                    pl.semaphore_signal(
                        credit_r.at[slot, t], inc=1,
                        device_id=(left,), device_id_type=pl.DeviceIdType.MESH,
                    )
                if l_active:
                    cur_l[t].wait_recv()
                    if s < N_STEP_L - 1:
                        val_l = (recv_l[slot, t, :, :].astype(jnp.float32)
                                 + p_l[:, cs])
                        cur_l[t].wait_send()
                        send_l[t, :, :] = val_l.astype(jnp.bfloat16)
                        if s + 1 >= 2:
                            pl.semaphore_wait(credit_l.at[nslot, t], 1)
                        cur_l[t] = make_rdma("l", nslot, t)
                        cur_l[t].start()
                    else:
                        final_l[t] = recv_l[slot, t, :, :].astype(jnp.float32)
                    if s <= N_STEP_L - 3:
                        pl.semaphore_signal(
                            credit_l.at[slot, t], inc=1,
                            device_id=(right,),
                            device_id_type=pl.DeviceIdType.MESH,
                        )

        for t in range(T):
            cur_r[t].wait_send()
            cur_l[t].wait_send()
            out_ref[:, t * tw:(t + 1) * tw] = _gelu(out_r[t] + final_l[t])

    out_shape = jax.ShapeDtypeStruct((m_chunk, n), jnp.float32)
    return pl.pallas_call(
        body,
        out_shape=out_shape,
        in_specs=[
            pl.BlockSpec(memory_space=pltpu.VMEM),
            pl.BlockSpec(memory_space=pltpu.VMEM),
        ],
        out_specs=pl.BlockSpec(memory_space=pltpu.VMEM),
        scratch_shapes=[
            pltpu.VMEM((2, T, m_chunk, tw), jnp.bfloat16),
            pltpu.VMEM((T, m_chunk, tw), jnp.bfloat16),
            pltpu.VMEM((2, T, m_chunk, tw), jnp.bfloat16),
            pltpu.VMEM((T, m_chunk, tw), jnp.bfloat16),
            pltpu.SemaphoreType.DMA((T,)),
            pltpu.SemaphoreType.DMA((2, T)),
            pltpu.SemaphoreType.DMA((T,)),
            pltpu.SemaphoreType.DMA((2, T)),
            pltpu.SemaphoreType.REGULAR((2, T)),
            pltpu.SemaphoreType.REGULAR((2, T)),
        ],
        compiler_params=pltpu.CompilerParams(collective_id=0),
    )(x, w_mat)
